# baseline (speedup 1.0000x reference)
import functools

import jax
import jax.numpy as jnp
from jax import lax
from jax.experimental import pallas as pl
from jax.experimental.pallas import tpu as pltpu
from jax.experimental.pallas import tpu_sc as plsc

TT = 5
S1 = 5
S2 = 2
NB = 4096
N1 = NB * S1
DF = 128
HID0 = 128
HID1 = 64
OUTC = 64
GROUPS = 8
TAPS = (0, 1, 3, 5)
KREAD = 5
VTH = 1.0

NC = 2
NS = 16
NW = NC * NS
CH = 128

_H0_PW = NB // NW
_H1_PW = N1 // NW
_NCH = _H1_PW // CH


def _sc_seed_body(x_hbm, nodes_hbm, h0_out, idx_v, buf, sem):
  wid = lax.axis_index("s") * NC + lax.axis_index("c")
  pltpu.sync_copy(nodes_hbm.at[pl.ds(wid * _H0_PW, CH)], idx_v)
  pltpu.async_copy(x_hbm.at[idx_v], buf, sem).wait()
  pltpu.sync_copy(buf, h0_out.at[pl.ds(wid * _H0_PW, CH)])


def _sc_step_body(x_hbm, h1i_hbm, h2e_hbm, h2o_hbm,
                  h1_out, m2_out,
                  idx_a, idx_b, idx_c, idx_d, buf_a, buf_b, buf_c, buf_d,
                  gsem_a, gsem_b, gsem_c, gsem_d, wsem_a, wsem_b):
  wid = lax.axis_index("s") * NC + lax.axis_index("c")

  idx = (idx_a, idx_b)
  bufs = (buf_a, buf_b)
  gsems = (gsem_a, gsem_b)
  wsems = (wsem_a, wsem_b)

  def _base(c):
    return wid * _H1_PW + c * CH

  gcp = [None, None]
  wcp = [None, None]
  for c in range(_NCH + 1):
    p = c % 2
    if c < _NCH:
      if wcp[p] is not None:
        wcp[p].wait()
      pltpu.sync_copy(h1i_hbm.at[pl.ds(_base(c), CH)], idx[p])
      gcp[p] = pltpu.async_copy(x_hbm.at[idx[p]], bufs[p], gsems[p])
    if c >= 1:
      q = (c - 1) % 2
      gcp[q].wait()
      wcp[q] = pltpu.async_copy(bufs[q], h1_out.at[pl.ds(_base(c - 1), CH)],
                                wsems[q])
  wcp[0].wait()
  wcp[1].wait()

  ebufs = (buf_a, buf_b)
  obufs = (buf_c, buf_d)
  oidx = (idx_c, idx_d)
  egsems = (gsem_a, gsem_b)
  ogsems = (gsem_c, gsem_d)
  gcp = [None, None]
  ocp = [None, None]
  wcp = [None, None]
  for c in range(_NCH + 1):
    p = c % 2
    if c < _NCH:
      if wcp[p] is not None:
        wcp[p].wait()
      pltpu.sync_copy(h2e_hbm.at[pl.ds(_base(c), CH)], idx[p])
      gcp[p] = pltpu.async_copy(x_hbm.at[idx[p]], ebufs[p], egsems[p])
      pltpu.sync_copy(h2o_hbm.at[pl.ds(_base(c), CH)], oidx[p])
      ocp[p] = pltpu.async_copy(x_hbm.at[oidx[p]], obufs[p], ogsems[p])
    if c >= 1:
      q = (c - 1) % 2
      gcp[q].wait()
      ocp[q].wait()
      ebuf, obuf = ebufs[q], obufs[q]

      def _add_row(r, _):
        for cc in range(DF // 16):
          sl = pl.ds(cc * 16, 16)
          ebuf[r, sl] = ebuf[r, sl] + obuf[r, sl]
        return 0

      lax.fori_loop(0, CH, _add_row, 0)
      wcp[q] = pltpu.async_copy(ebuf, m2_out.at[pl.ds(_base(c - 1), CH)],
                                wsems[q])
  wcp[0].wait()
  wcp[1].wait()


def _sc_mesh():
  return plsc.VectorSubcoreMesh(core_axis_name="c", subcore_axis_name="s",
                                num_cores=NC, num_subcores=NS)


@functools.cache
def _sc_seed_fn():
  return pl.kernel(
      _sc_seed_body,
      out_type=jax.ShapeDtypeStruct((NB, DF), jnp.float32),
      mesh=_sc_mesh(),
      scratch_types=[
          pltpu.VMEM((CH,), jnp.int32),
          pltpu.VMEM((CH, DF), jnp.float32),
          pltpu.SemaphoreType.DMA,
      ],
  )


@functools.cache
def _sc_step_fn():
  return pl.kernel(
      _sc_step_body,
      out_type=(
          jax.ShapeDtypeStruct((N1, DF), jnp.float32),
          jax.ShapeDtypeStruct((N1, DF), jnp.float32),
      ),
      mesh=_sc_mesh(),
      scratch_types=[
          pltpu.VMEM((CH,), jnp.int32),
          pltpu.VMEM((CH,), jnp.int32),
          pltpu.VMEM((CH,), jnp.int32),
          pltpu.VMEM((CH,), jnp.int32),
          pltpu.VMEM((CH, DF), jnp.float32),
          pltpu.VMEM((CH, DF), jnp.float32),
          pltpu.VMEM((CH, DF), jnp.float32),
          pltpu.VMEM((CH, DF), jnp.float32),
          pltpu.SemaphoreType.DMA,
          pltpu.SemaphoreType.DMA,
          pltpu.SemaphoreType.DMA,
          pltpu.SemaphoreType.DMA,
          pltpu.SemaphoreType.DMA,
          pltpu.SemaphoreType.DMA,
      ],
  )


NSB = 4
SBB = NB // NSB
SBN1 = SBB * S1


def _step_net(t, h0_ref, h1_ref, m2_ref, wl0_ref, wr0_ref, b0_ref,
              wl1_ref, wr1_ref, b1_ref, dwt_ref, kdt_ref):
  f32 = jnp.float32

  h1b = h1_ref[...]
  m2s = m2_ref[...]
  wl0 = wl0_ref[...]
  wr0 = wr0_ref[...]
  b0 = b0_ref[...]

  m1 = (h1b[0] + h1b[1] + h1b[2] + h1b[3] + h1b[4]) * (1.0 / S1)
  a_top = jnp.dot(h0_ref[...], wl0, preferred_element_type=f32) \
      + jnp.dot(m1, wr0, preferred_element_type=f32) + b0
  a_bot = jnp.dot(h1b.reshape(SBN1, DF), wl0, preferred_element_type=f32) \
      + jnp.dot(m2s.reshape(SBN1, DF), wr0 * 0.5,
                preferred_element_type=f32) + b0
  s_top = (a_top >= VTH).astype(f32)
  s_bot = (a_bot >= VTH).astype(f32).reshape(S1, SBB, HID0)

  g1m = (s_bot[0] + s_bot[1] + s_bot[2] + s_bot[3] + s_bot[4]) * (1.0 / S1)
  a1 = jnp.dot(s_top, wl1_ref[...], preferred_element_type=f32) \
      + jnp.dot(g1m, wr1_ref[...], preferred_element_type=f32) + b1_ref[...]
  s1 = (a1 >= VTH).astype(f32)

  dwt = dwt_ref[...]
  kdt = kdt_ref[...]
  e = jnp.exp(dwt)
  gw = e / jnp.sum(e, axis=0, keepdims=True)
  a_rows = [
      jnp.sum(kdt[0:3], axis=0, keepdims=True),
      jnp.sum(kdt[0:4], axis=0, keepdims=True),
      jnp.sum(kdt[0:5], axis=0, keepdims=True),
      jnp.sum(kdt[1:5], axis=0, keepdims=True),
      jnp.sum(kdt[2:5], axis=0, keepdims=True),
  ]
  ct = jnp.zeros((1, HID1), dtype=f32)
  for i, d in enumerate(TAPS):
    u = t + d
    if u < TT:
      ct = ct + gw[i:i + 1] * a_rows[u]
  return s1 * (ct * (1.0 / TT))


def _tc_first_body(h0_ref, h1_ref, m2_ref, wl0_ref, wr0_ref, b0_ref,
                   wl1_ref, wr1_ref, b1_ref, dwt_ref, kdt_ref, acc_ref):
  acc_ref[...] = _step_net(0, h0_ref, h1_ref, m2_ref, wl0_ref, wr0_ref,
                           b0_ref, wl1_ref, wr1_ref, b1_ref, dwt_ref,
                           kdt_ref)


def _tc_mid_body(t, h0_ref, h1_ref, m2_ref, wl0_ref, wr0_ref, b0_ref,
                 wl1_ref, wr1_ref, b1_ref, dwt_ref, kdt_ref, accin_ref,
                 acc_ref):
  acc_ref[...] = accin_ref[...] + _step_net(
      t, h0_ref, h1_ref, m2_ref, wl0_ref, wr0_ref, b0_ref, wl1_ref,
      wr1_ref, b1_ref, dwt_ref, kdt_ref)


def _tc_last_body(h0_ref, h1_ref, m2_ref, wl0_ref, wr0_ref, b0_ref,
                  wl1_ref, wr1_ref, b1_ref, dwt_ref, kdt_ref, accin_ref,
                  wp_ref, bp_ref, out_ref):
  acc = accin_ref[...] + _step_net(
      TT - 1, h0_ref, h1_ref, m2_ref, wl0_ref, wr0_ref, b0_ref, wl1_ref,
      wr1_ref, b1_ref, dwt_ref, kdt_ref)
  out_ref[...] = jnp.dot(acc, wp_ref[...],
                         preferred_element_type=jnp.float32) + bp_ref[...]


def _full(shape):
  return pl.BlockSpec(shape, lambda sb: (0,) * len(shape))


_STEP_SPECS = [
    pl.BlockSpec((SBB, DF), lambda sb: (sb, 0)),
    pl.BlockSpec((S1, SBB, DF), lambda sb: (0, sb, 0)),
    pl.BlockSpec((S1, SBB, DF), lambda sb: (0, sb, 0)),
    _full((DF, HID0)),
    _full((DF, HID0)),
    _full((1, HID0)),
    _full((HID0, HID1)),
    _full((HID0, HID1)),
    _full((1, HID1)),
    _full((len(TAPS), HID1)),
    _full((KREAD, HID1)),
]

_ACC_SPEC = pl.BlockSpec((SBB, HID1), lambda sb: (sb, 0))


def _tc_step(t, h0g, h1g, m2g, weights, acc, wp, bpr):
  args = [h0g, h1g, m2g] + list(weights)
  if t == 0:
    return pl.pallas_call(
        _tc_first_body,
        grid=(NSB,),
        in_specs=list(_STEP_SPECS),
        out_specs=_ACC_SPEC,
        out_shape=jax.ShapeDtypeStruct((NB, HID1), jnp.float32),
    )(*args)
  if t < TT - 1:
    return pl.pallas_call(
        functools.partial(_tc_mid_body, t),
        grid=(NSB,),
        in_specs=list(_STEP_SPECS) + [_ACC_SPEC],
        out_specs=_ACC_SPEC,
        out_shape=jax.ShapeDtypeStruct((NB, HID1), jnp.float32),
    )(*args, acc)
  return pl.pallas_call(
      _tc_last_body,
      grid=(NSB,),
      in_specs=list(_STEP_SPECS) + [_ACC_SPEC,
                                    _full((HID1, OUTC)), _full((1, OUTC))],
      out_specs=pl.BlockSpec((SBB, OUTC), lambda sb: (sb, 0)),
      out_shape=jax.ShapeDtypeStruct((NB, OUTC), jnp.float32),
  )(*args, acc, wp, bpr)


def kernel(x, nodes, hop1, hop2, Wl0, Wr0, b0, Wl1, Wr1, b1,
           delay_w, k_dw, Wp, bp):
  h1i = hop1.transpose(0, 2, 1).reshape(TT, N1)
  h2 = hop2.reshape(TT, NB, S1, S2).transpose(0, 2, 1, 3).reshape(TT, N1, S2)
  h2e = h2[:, :, 0]
  h2o = h2[:, :, 1]

  h0g = _sc_seed_fn()(x, nodes)

  dwt = jnp.repeat(delay_w, HID1 // GROUPS, axis=0).T
  kdt = k_dw.T
  weights = (Wl0, Wr0, b0.reshape(1, -1), Wl1, Wr1, b1.reshape(1, -1),
             dwt, kdt)

  gathered = [_sc_step_fn()(x, h1i[t], h2e[t], h2o[t]) for t in range(2)]
  acc = None
  for t in range(TT):
    if t + 2 < TT:
      gathered.append(_sc_step_fn()(x, h1i[t + 2], h2e[t + 2], h2o[t + 2]))
    h1g, m2g = gathered[t]
    h1g = h1g.reshape(S1, NB, DF)
    m2g = m2g.reshape(S1, NB, DF)
    acc = _tc_step(t, h0g, h1g, m2g, weights, acc,
                   Wp, bp.reshape(1, -1))
  return acc


# speedup vs baseline: 1.0600x; 1.0600x over previous
import functools

import jax
import jax.numpy as jnp
from jax import lax
from jax.experimental import pallas as pl
from jax.experimental.pallas import tpu as pltpu
from jax.experimental.pallas import tpu_sc as plsc

TT = 5
S1 = 5
S2 = 2
NB = 4096
N1 = NB * S1
DF = 128
HID0 = 128
HID1 = 64
OUTC = 64
GROUPS = 8
TAPS = (0, 1, 3, 5)
KREAD = 5
VTH = 1.0

NC = 2
NS = 16
NW = NC * NS
CH = 128

_H0_PW = NB // NW
_H1_PW = N1 // NW
_NCH = _H1_PW // CH


_NBUF = 6
_D1 = 4
_D2 = 3


def _sc_body(x_hbm, nodes_hbm, h1i_hbm, h2e_hbm, h2o_hbm,
             h0_out, h1_out, m2_out, *scratch):
  idx = scratch[0:8]
  bufs = scratch[8:8 + _NBUF]
  gsems = scratch[8 + _NBUF:16 + _NBUF]
  wsems = scratch[16 + _NBUF:20 + _NBUF]
  wid = lax.axis_index("s") * NC + lax.axis_index("c")

  pltpu.sync_copy(nodes_hbm.at[pl.ds(wid * _H0_PW, CH)], idx[0])
  pltpu.async_copy(x_hbm.at[idx[0]], bufs[0], gsems[0]).wait()
  pltpu.sync_copy(bufs[0], h0_out.at[pl.ds(wid * _H0_PW, CH)])

  n_chunks = TT * _NCH

  def _src(c):
    t, k = divmod(c, _NCH)
    return t * N1 + wid * _H1_PW + k * CH

  def _dst(c):
    t, k = divmod(c, _NCH)
    return t, wid * _H1_PW + k * CH

  gcp = [None] * _D1
  wcp = [None] * _D1
  for c in range(n_chunks + _D1 - 1):
    p = c % _D1
    if c < n_chunks:
      if wcp[p] is not None:
        wcp[p].wait()
      pltpu.sync_copy(h1i_hbm.at[pl.ds(_src(c), CH)], idx[p])
      gcp[p] = pltpu.async_copy(x_hbm.at[idx[p]], bufs[p], gsems[p])
    d = c - (_D1 - 1)
    if d >= 0:
      q = d % _D1
      gcp[q].wait()
      t, base = _dst(d)
      wcp[q] = pltpu.async_copy(bufs[q], h1_out.at[t, pl.ds(base, CH)],
                                wsems[q])
  for q in range(_D1):
    if wcp[q] is not None:
      wcp[q].wait()

  ebufs = bufs[0:_D2]
  obufs = bufs[_D2:2 * _D2]
  eidx = idx[0:_D2]
  oidx = idx[_D2:2 * _D2]
  egsems = gsems[0:_D2]
  ogsems = gsems[_D2:2 * _D2]
  gcp = [None] * _D2
  ocp = [None] * _D2
  wcp = [None] * _D2
  for c in range(n_chunks + _D2 - 1):
    p = c % _D2
    if c < n_chunks:
      if wcp[p] is not None:
        wcp[p].wait()
      pltpu.sync_copy(h2e_hbm.at[pl.ds(_src(c), CH)], eidx[p])
      gcp[p] = pltpu.async_copy(x_hbm.at[eidx[p]], ebufs[p], egsems[p])
      pltpu.sync_copy(h2o_hbm.at[pl.ds(_src(c), CH)], oidx[p])
      ocp[p] = pltpu.async_copy(x_hbm.at[oidx[p]], obufs[p], ogsems[p])
    d = c - (_D2 - 1)
    if d >= 0:
      q = d % _D2
      gcp[q].wait()
      ocp[q].wait()
      ebuf, obuf = ebufs[q], obufs[q]

      def _add_row(r, _):
        for cc in range(DF // 16):
          sl = pl.ds(cc * 16, 16)
          ebuf[r, sl] = ebuf[r, sl] + obuf[r, sl]
        return 0

      lax.fori_loop(0, CH, _add_row, 0)
      t, base = _dst(d)
      wcp[q] = pltpu.async_copy(ebuf, m2_out.at[t, pl.ds(base, CH)],
                                wsems[q])
  for q in range(_D2):
    if wcp[q] is not None:
      wcp[q].wait()


@functools.cache
def _sc_gather_fn():
  return pl.kernel(
      _sc_body,
      out_type=(
          jax.ShapeDtypeStruct((NB, DF), jnp.float32),
          jax.ShapeDtypeStruct((TT, N1, DF), jnp.float32),
          jax.ShapeDtypeStruct((TT, N1, DF), jnp.float32),
      ),
      mesh=plsc.VectorSubcoreMesh(core_axis_name="c", subcore_axis_name="s",
                                  num_cores=NC, num_subcores=NS),
      scratch_types=(
          [pltpu.VMEM((CH,), jnp.int32)] * 8
          + [pltpu.VMEM((CH, DF), jnp.float32)] * _NBUF
          + [pltpu.SemaphoreType.DMA] * 12
      ),
  )


NSB = 4
SBB = NB // NSB
SBN1 = SBB * S1


def _tc_body(h0_ref, h1_ref, m2_ref, wl0_ref, wr0_ref, b0_ref,
             wl1_ref, wr1_ref, b1_ref, dwt_ref, kdt_ref, wp_ref, bp_ref,
             out_ref, acc_ref):
  sb = pl.program_id(0)
  t = pl.program_id(1)
  del sb
  f32 = jnp.float32

  h1b = h1_ref[0]
  m2s = m2_ref[0]
  wl0 = wl0_ref[...]
  wr0 = wr0_ref[...]
  b0 = b0_ref[...]

  m1 = (h1b[0] + h1b[1] + h1b[2] + h1b[3] + h1b[4]) * (1.0 / S1)
  a_top = jnp.dot(h0_ref[...], wl0, preferred_element_type=f32) \
      + jnp.dot(m1, wr0, preferred_element_type=f32) + b0
  a_bot = jnp.dot(h1b.reshape(SBN1, DF), wl0, preferred_element_type=f32) \
      + jnp.dot(m2s.reshape(SBN1, DF), wr0 * 0.5,
                preferred_element_type=f32) + b0
  s_top = (a_top >= VTH).astype(f32)
  s_bot = (a_bot >= VTH).astype(f32).reshape(S1, SBB, HID0)

  g1m = (s_bot[0] + s_bot[1] + s_bot[2] + s_bot[3] + s_bot[4]) * (1.0 / S1)
  a1 = jnp.dot(s_top, wl1_ref[...], preferred_element_type=f32) \
      + jnp.dot(g1m, wr1_ref[...], preferred_element_type=f32) + b1_ref[...]
  s1 = (a1 >= VTH).astype(f32)

  dwt = dwt_ref[...]
  kdt = kdt_ref[...]
  e = jnp.exp(dwt)
  gw = e / jnp.sum(e, axis=0, keepdims=True)
  a_rows = [
      jnp.sum(kdt[0:3], axis=0, keepdims=True),
      jnp.sum(kdt[0:4], axis=0, keepdims=True),
      jnp.sum(kdt[0:5], axis=0, keepdims=True),
      jnp.sum(kdt[1:5], axis=0, keepdims=True),
      jnp.sum(kdt[2:5], axis=0, keepdims=True),
  ]
  ct = jnp.zeros((1, HID1), dtype=f32)
  for tp in range(TT):
    row = jnp.zeros((1, HID1), dtype=f32)
    for i, d in enumerate(TAPS):
      u = tp + d
      if u < TT:
        row = row + gw[i:i + 1] * a_rows[u]
    sel = jnp.where(t == tp, 1.0 / TT, 0.0).astype(f32)
    ct = ct + sel * row

  @pl.when(t == 0)
  def _():
    acc_ref[...] = jnp.zeros_like(acc_ref)

  acc_ref[...] = acc_ref[...] + s1 * ct

  @pl.when(t == TT - 1)
  def _():
    out_ref[...] = jnp.dot(acc_ref[...], wp_ref[...],
                           preferred_element_type=f32) + bp_ref[...]


def _tc_net(h0g, h1g, m2g, wl0, wr0, b0r, wl1, wr1, b1r, dwt, kdt, wp, bpr):
  full = lambda shape: pl.BlockSpec(shape, lambda sb, t: (0,) * len(shape))
  return pl.pallas_call(
      _tc_body,
      grid=(NSB, TT),
      in_specs=[
          pl.BlockSpec((SBB, DF), lambda sb, t: (sb, 0)),
          pl.BlockSpec((1, S1, SBB, DF), lambda sb, t: (t, 0, sb, 0)),
          pl.BlockSpec((1, S1, SBB, DF), lambda sb, t: (t, 0, sb, 0)),
          full((DF, HID0)),
          full((DF, HID0)),
          full((1, HID0)),
          full((HID0, HID1)),
          full((HID0, HID1)),
          full((1, HID1)),
          full((len(TAPS), HID1)),
          full((KREAD, HID1)),
          full((HID1, OUTC)),
          full((1, OUTC)),
      ],
      out_specs=pl.BlockSpec((SBB, OUTC), lambda sb, t: (sb, 0)),
      out_shape=jax.ShapeDtypeStruct((NB, OUTC), jnp.float32),
      scratch_shapes=[pltpu.VMEM((SBB, HID1), jnp.float32)],
  )(h0g, h1g, m2g, wl0, wr0, b0r, wl1, wr1, b1r, dwt, kdt, wp, bpr)


def kernel(x, nodes, hop1, hop2, Wl0, Wr0, b0, Wl1, Wr1, b1,
           delay_w, k_dw, Wp, bp):
  h1i = hop1.transpose(0, 2, 1).reshape(TT * N1)
  h2 = hop2.reshape(TT, NB, S1, S2).transpose(0, 2, 1, 3).reshape(TT * N1, S2)
  h2e = h2[:, 0]
  h2o = h2[:, 1]
  h0g, h1g, m2g = _sc_gather_fn()(x, nodes, h1i, h2e, h2o)
  h1g = h1g.reshape(TT, S1, NB, DF)
  m2g = m2g.reshape(TT, S1, NB, DF)
  dwt = jnp.repeat(delay_w, HID1 // GROUPS, axis=0).T
  kdt = k_dw.T
  return _tc_net(h0g, h1g, m2g, Wl0, Wr0, b0.reshape(1, -1),
                 Wl1, Wr1, b1.reshape(1, -1), dwt, kdt,
                 Wp, bp.reshape(1, -1))


# speedup vs baseline: 1.1216x; 1.0581x over previous
import functools

import jax
import jax.numpy as jnp
from jax import lax
from jax.experimental import pallas as pl
from jax.experimental.pallas import tpu as pltpu
from jax.experimental.pallas import tpu_sc as plsc

TT = 5
S1 = 5
S2 = 2
NB = 4096
N1 = NB * S1
DF = 128
HID0 = 128
HID1 = 64
OUTC = 64
GROUPS = 8
TAPS = (0, 1, 3, 5)
KREAD = 5
VTH = 1.0

NC = 2
NS = 16
NW = NC * NS
CH = 128

_H0_PW = NB // NW
_H1_PW = N1 // NW
_NCH = _H1_PW // CH


_NBUF = 6
_D1 = 4
_D2 = 3


def _sc_body(x_hbm, nodes_hbm, h1i_hbm, h2e_hbm, h2o_hbm,
             h0_out, h1_out, m2_out, *scratch):
  idx = scratch[0:8]
  bufs = scratch[8:8 + _NBUF]
  gsems = scratch[8 + _NBUF:16 + _NBUF]
  wsems = scratch[16 + _NBUF:20 + _NBUF]
  wid = lax.axis_index("s") * NC + lax.axis_index("c")

  pltpu.sync_copy(nodes_hbm.at[pl.ds(wid * _H0_PW, CH)], idx[0])
  pltpu.async_copy(x_hbm.at[idx[0]], bufs[0], gsems[0]).wait()
  pltpu.sync_copy(bufs[0], h0_out.at[pl.ds(wid * _H0_PW, CH)])

  n_chunks = TT * _NCH

  def _src(c):
    t, k = divmod(c, _NCH)
    return t * N1 + wid * _H1_PW + k * CH

  def _dst(c):
    t, k = divmod(c, _NCH)
    return t, wid * _H1_PW + k * CH

  hbufs = bufs[0:2]
  ebufs = bufs[2:4]
  obufs = bufs[4:6]
  hidx = idx[0:2]
  eidx = idx[2:4]
  oidx = idx[4:6]
  hsems = gsems[0:2]
  esems = gsems[2:4]
  osems = gsems[4:6]
  hwsems = wsems[0:2]
  mwsems = wsems[2:4]
  g1 = [None, None]
  g2 = [None, None]
  g3 = [None, None]
  wh = [None, None]
  wm = [None, None]
  for c in range(n_chunks + 1):
    p = c % 2
    if c < n_chunks:
      if wh[p] is not None:
        wh[p].wait()
        wm[p].wait()
      pltpu.sync_copy(h1i_hbm.at[pl.ds(_src(c), CH)], hidx[p])
      g1[p] = pltpu.async_copy(x_hbm.at[hidx[p]], hbufs[p], hsems[p])
      pltpu.sync_copy(h2e_hbm.at[pl.ds(_src(c), CH)], eidx[p])
      g2[p] = pltpu.async_copy(x_hbm.at[eidx[p]], ebufs[p], esems[p])
      pltpu.sync_copy(h2o_hbm.at[pl.ds(_src(c), CH)], oidx[p])
      g3[p] = pltpu.async_copy(x_hbm.at[oidx[p]], obufs[p], osems[p])
    if c >= 1:
      q = (c - 1) % 2
      t, base = _dst(c - 1)
      g1[q].wait()
      wh[q] = pltpu.async_copy(hbufs[q], h1_out.at[t, pl.ds(base, CH)],
                               hwsems[q])
      g2[q].wait()
      g3[q].wait()
      ebuf, obuf = ebufs[q], obufs[q]

      def _add_row(r, _):
        for cc in range(DF // 16):
          sl = pl.ds(cc * 16, 16)
          ebuf[r, sl] = ebuf[r, sl] + obuf[r, sl]
        return 0

      lax.fori_loop(0, CH, _add_row, 0)
      wm[q] = pltpu.async_copy(ebuf, m2_out.at[t, pl.ds(base, CH)],
                               mwsems[q])
  for q in range(2):
    if wh[q] is not None:
      wh[q].wait()
      wm[q].wait()


@functools.cache
def _sc_gather_fn():
  return pl.kernel(
      _sc_body,
      out_type=(
          jax.ShapeDtypeStruct((NB, DF), jnp.float32),
          jax.ShapeDtypeStruct((TT, N1, DF), jnp.float32),
          jax.ShapeDtypeStruct((TT, N1, DF), jnp.float32),
      ),
      mesh=plsc.VectorSubcoreMesh(core_axis_name="c", subcore_axis_name="s",
                                  num_cores=NC, num_subcores=NS),
      scratch_types=(
          [pltpu.VMEM((CH,), jnp.int32)] * 8
          + [pltpu.VMEM((CH, DF), jnp.float32)] * _NBUF
          + [pltpu.SemaphoreType.DMA] * 12
      ),
  )


NSB = 4
SBB = NB // NSB
SBN1 = SBB * S1


def _tc_body(h0_ref, h1_ref, m2_ref, wl0_ref, wr0_ref, b0_ref,
             wl1_ref, wr1_ref, b1_ref, dwt_ref, kdt_ref, wp_ref, bp_ref,
             out_ref, acc_ref):
  sb = pl.program_id(0)
  t = pl.program_id(1)
  del sb
  f32 = jnp.float32

  h1b = h1_ref[0]
  m2s = m2_ref[0]
  wl0 = wl0_ref[...]
  wr0 = wr0_ref[...]
  b0 = b0_ref[...]

  m1 = (h1b[0] + h1b[1] + h1b[2] + h1b[3] + h1b[4]) * (1.0 / S1)
  a_top = jnp.dot(h0_ref[...], wl0, preferred_element_type=f32) \
      + jnp.dot(m1, wr0, preferred_element_type=f32) + b0
  a_bot = jnp.dot(h1b.reshape(SBN1, DF), wl0, preferred_element_type=f32) \
      + jnp.dot(m2s.reshape(SBN1, DF), wr0 * 0.5,
                preferred_element_type=f32) + b0
  s_top = (a_top >= VTH).astype(f32)
  s_bot = (a_bot >= VTH).astype(f32).reshape(S1, SBB, HID0)

  g1m = (s_bot[0] + s_bot[1] + s_bot[2] + s_bot[3] + s_bot[4]) * (1.0 / S1)
  a1 = jnp.dot(s_top, wl1_ref[...], preferred_element_type=f32) \
      + jnp.dot(g1m, wr1_ref[...], preferred_element_type=f32) + b1_ref[...]
  s1 = (a1 >= VTH).astype(f32)

  dwt = dwt_ref[...]
  kdt = kdt_ref[...]
  e = jnp.exp(dwt)
  gw = e / jnp.sum(e, axis=0, keepdims=True)
  a_rows = [
      jnp.sum(kdt[0:3], axis=0, keepdims=True),
      jnp.sum(kdt[0:4], axis=0, keepdims=True),
      jnp.sum(kdt[0:5], axis=0, keepdims=True),
      jnp.sum(kdt[1:5], axis=0, keepdims=True),
      jnp.sum(kdt[2:5], axis=0, keepdims=True),
  ]
  ct = jnp.zeros((1, HID1), dtype=f32)
  for tp in range(TT):
    row = jnp.zeros((1, HID1), dtype=f32)
    for i, d in enumerate(TAPS):
      u = tp + d
      if u < TT:
        row = row + gw[i:i + 1] * a_rows[u]
    sel = jnp.where(t == tp, 1.0 / TT, 0.0).astype(f32)
    ct = ct + sel * row

  @pl.when(t == 0)
  def _():
    acc_ref[...] = jnp.zeros_like(acc_ref)

  acc_ref[...] = acc_ref[...] + s1 * ct

  @pl.when(t == TT - 1)
  def _():
    out_ref[...] = jnp.dot(acc_ref[...], wp_ref[...],
                           preferred_element_type=f32) + bp_ref[...]


def _tc_net(h0g, h1g, m2g, wl0, wr0, b0r, wl1, wr1, b1r, dwt, kdt, wp, bpr):
  full = lambda shape: pl.BlockSpec(shape, lambda sb, t: (0,) * len(shape))
  return pl.pallas_call(
      _tc_body,
      grid=(NSB, TT),
      in_specs=[
          pl.BlockSpec((SBB, DF), lambda sb, t: (sb, 0)),
          pl.BlockSpec((1, S1, SBB, DF), lambda sb, t: (t, 0, sb, 0)),
          pl.BlockSpec((1, S1, SBB, DF), lambda sb, t: (t, 0, sb, 0)),
          full((DF, HID0)),
          full((DF, HID0)),
          full((1, HID0)),
          full((HID0, HID1)),
          full((HID0, HID1)),
          full((1, HID1)),
          full((len(TAPS), HID1)),
          full((KREAD, HID1)),
          full((HID1, OUTC)),
          full((1, OUTC)),
      ],
      out_specs=pl.BlockSpec((SBB, OUTC), lambda sb, t: (sb, 0)),
      out_shape=jax.ShapeDtypeStruct((NB, OUTC), jnp.float32),
      scratch_shapes=[pltpu.VMEM((SBB, HID1), jnp.float32)],
  )(h0g, h1g, m2g, wl0, wr0, b0r, wl1, wr1, b1r, dwt, kdt, wp, bpr)


def kernel(x, nodes, hop1, hop2, Wl0, Wr0, b0, Wl1, Wr1, b1,
           delay_w, k_dw, Wp, bp):
  h1i = hop1.transpose(0, 2, 1).reshape(TT * N1)
  h2 = hop2.reshape(TT, NB, S1, S2).transpose(0, 2, 1, 3).reshape(TT * N1, S2)
  h2e = h2[:, 0]
  h2o = h2[:, 1]
  h0g, h1g, m2g = _sc_gather_fn()(x, nodes, h1i, h2e, h2o)
  h1g = h1g.reshape(TT, S1, NB, DF)
  m2g = m2g.reshape(TT, S1, NB, DF)
  dwt = jnp.repeat(delay_w, HID1 // GROUPS, axis=0).T
  kdt = k_dw.T
  return _tc_net(h0g, h1g, m2g, Wl0, Wr0, b0.reshape(1, -1),
                 Wl1, Wr1, b1.reshape(1, -1), dwt, kdt,
                 Wp, bp.reshape(1, -1))


# speedup vs baseline: 1.1488x; 1.0243x over previous
import functools

import jax
import jax.numpy as jnp
from jax import lax
from jax.experimental import pallas as pl
from jax.experimental.pallas import tpu as pltpu
from jax.experimental.pallas import tpu_sc as plsc

TT = 5
S1 = 5
S2 = 2
NB = 4096
N1 = NB * S1
DF = 128
HID0 = 128
HID1 = 64
OUTC = 64
GROUPS = 8
TAPS = (0, 1, 3, 5)
KREAD = 5
VTH = 1.0

NC = 2
NS = 16
NW = NC * NS
CH = 128

_H0_PW = NB // NW
_H1_PW = N1 // NW
_NCH = _H1_PW // CH


_NBUF = 6
_D1 = 4
_D2 = 3


def _sc_body(x_hbm, nodes_hbm, h1i_hbm, h2e_hbm, h2o_hbm,
             h0_out, h1_out, m2_out, *scratch):
  idx = scratch[0:8]
  bufs = scratch[8:8 + _NBUF]
  gsems = scratch[8 + _NBUF:16 + _NBUF]
  wsems = scratch[16 + _NBUF:20 + _NBUF]
  wid = lax.axis_index("s") * NC + lax.axis_index("c")

  pltpu.sync_copy(nodes_hbm.at[pl.ds(wid * _H0_PW, CH)], idx[0])
  pltpu.async_copy(x_hbm.at[idx[0]], bufs[0], gsems[0]).wait()
  pltpu.sync_copy(bufs[0], h0_out.at[pl.ds(wid * _H0_PW, CH)])

  n_chunks = TT * _NCH

  def _src(c):
    t, k = divmod(c, _NCH)
    return t * N1 + wid * _H1_PW + k * CH

  def _dst(c):
    t, k = divmod(c, _NCH)
    return t, wid * _H1_PW + k * CH

  hbufs = bufs[0:2]
  ebufs = bufs[2:4]
  obufs = bufs[4:6]
  hidx = idx[0:2]
  eidx = idx[2:4]
  oidx = idx[4:6]
  hsems = gsems[0:2]
  esems = gsems[2:4]
  osems = gsems[4:6]
  hwsems = wsems[0:2]
  mwsems = wsems[2:4]
  g1 = [None, None]
  g2 = [None, None]
  g3 = [None, None]
  wh = [None, None]
  wm = [None, None]
  for c in range(n_chunks + 1):
    p = c % 2
    if c < n_chunks:
      if wh[p] is not None:
        wh[p].wait()
        wm[p].wait()
      pltpu.sync_copy(h1i_hbm.at[pl.ds(_src(c), CH)], hidx[p])
      g1[p] = pltpu.async_copy(x_hbm.at[hidx[p]], hbufs[p], hsems[p])
      pltpu.sync_copy(h2e_hbm.at[pl.ds(_src(c), CH)], eidx[p])
      g2[p] = pltpu.async_copy(x_hbm.at[eidx[p]], ebufs[p], esems[p])
      pltpu.sync_copy(h2o_hbm.at[pl.ds(_src(c), CH)], oidx[p])
      g3[p] = pltpu.async_copy(x_hbm.at[oidx[p]], obufs[p], osems[p])
    if c >= 1:
      q = (c - 1) % 2
      t, base = _dst(c - 1)
      g1[q].wait()
      wh[q] = pltpu.async_copy(hbufs[q], h1_out.at[t, pl.ds(base, CH)],
                               hwsems[q])
      g2[q].wait()
      g3[q].wait()
      ebuf, obuf = ebufs[q], obufs[q]

      def _add_row(r, _):
        for cc in range(DF // 16):
          sl = pl.ds(cc * 16, 16)
          ebuf[r, sl] = ebuf[r, sl] + obuf[r, sl]
        return 0

      lax.fori_loop(0, CH, _add_row, 0)
      wm[q] = pltpu.async_copy(ebuf, m2_out.at[t, pl.ds(base, CH)],
                               mwsems[q])
  for q in range(2):
    if wh[q] is not None:
      wh[q].wait()
      wm[q].wait()


@functools.cache
def _sc_gather_fn():
  return pl.kernel(
      _sc_body,
      out_type=(
          jax.ShapeDtypeStruct((NB, DF), jnp.float32),
          jax.ShapeDtypeStruct((TT, N1, DF), jnp.float32),
          jax.ShapeDtypeStruct((TT, N1, DF), jnp.float32),
      ),
      mesh=plsc.VectorSubcoreMesh(core_axis_name="c", subcore_axis_name="s",
                                  num_cores=NC, num_subcores=NS),
      scratch_types=(
          [pltpu.VMEM((CH,), jnp.int32)] * 8
          + [pltpu.VMEM((CH, DF), jnp.float32)] * _NBUF
          + [pltpu.SemaphoreType.DMA] * 12
      ),
  )


NSB = 2
SBB = NB // NSB
SBN1 = SBB * S1


def _tc_body(h0_ref, h1_ref, m2_ref, wl0_ref, wr0_ref, b0_ref,
             wl1_ref, wr1_ref, b1_ref, dwt_ref, kdt_ref, wp_ref, bp_ref,
             out_ref, acc_ref):
  sb = pl.program_id(0)
  t = pl.program_id(1)
  del sb
  f32 = jnp.float32

  h1b = h1_ref[0]
  m2s = m2_ref[0]
  wl0 = wl0_ref[...]
  wr0 = wr0_ref[...]
  b0 = b0_ref[...]

  m1 = (h1b[0] + h1b[1] + h1b[2] + h1b[3] + h1b[4]) * (1.0 / S1)
  a_top = jnp.dot(h0_ref[...], wl0, preferred_element_type=f32) \
      + jnp.dot(m1, wr0, preferred_element_type=f32) + b0
  a_bot = jnp.dot(h1b.reshape(SBN1, DF), wl0, preferred_element_type=f32) \
      + jnp.dot(m2s.reshape(SBN1, DF), wr0 * 0.5,
                preferred_element_type=f32) + b0
  s_top = (a_top >= VTH).astype(f32)
  s_bot = (a_bot >= VTH).astype(f32).reshape(S1, SBB, HID0)

  g1m = (s_bot[0] + s_bot[1] + s_bot[2] + s_bot[3] + s_bot[4]) * (1.0 / S1)
  a1 = jnp.dot(s_top, wl1_ref[...], preferred_element_type=f32) \
      + jnp.dot(g1m, wr1_ref[...], preferred_element_type=f32) + b1_ref[...]
  s1 = (a1 >= VTH).astype(f32)

  dwt = dwt_ref[...]
  kdt = kdt_ref[...]
  e = jnp.exp(dwt)
  gw = e / jnp.sum(e, axis=0, keepdims=True)
  a_rows = [
      jnp.sum(kdt[0:3], axis=0, keepdims=True),
      jnp.sum(kdt[0:4], axis=0, keepdims=True),
      jnp.sum(kdt[0:5], axis=0, keepdims=True),
      jnp.sum(kdt[1:5], axis=0, keepdims=True),
      jnp.sum(kdt[2:5], axis=0, keepdims=True),
  ]
  ct = jnp.zeros((1, HID1), dtype=f32)
  for tp in range(TT):
    row = jnp.zeros((1, HID1), dtype=f32)
    for i, d in enumerate(TAPS):
      u = tp + d
      if u < TT:
        row = row + gw[i:i + 1] * a_rows[u]
    sel = jnp.where(t == tp, 1.0 / TT, 0.0).astype(f32)
    ct = ct + sel * row

  @pl.when(t == 0)
  def _():
    acc_ref[...] = jnp.zeros_like(acc_ref)

  acc_ref[...] = acc_ref[...] + s1 * ct

  @pl.when(t == TT - 1)
  def _():
    out_ref[...] = jnp.dot(acc_ref[...], wp_ref[...],
                           preferred_element_type=f32) + bp_ref[...]


def _tc_net(h0g, h1g, m2g, wl0, wr0, b0r, wl1, wr1, b1r, dwt, kdt, wp, bpr):
  full = lambda shape: pl.BlockSpec(shape, lambda sb, t: (0,) * len(shape))
  return pl.pallas_call(
      _tc_body,
      grid=(NSB, TT),
      in_specs=[
          pl.BlockSpec((SBB, DF), lambda sb, t: (sb, 0)),
          pl.BlockSpec((1, S1, SBB, DF), lambda sb, t: (t, 0, sb, 0)),
          pl.BlockSpec((1, S1, SBB, DF), lambda sb, t: (t, 0, sb, 0)),
          full((DF, HID0)),
          full((DF, HID0)),
          full((1, HID0)),
          full((HID0, HID1)),
          full((HID0, HID1)),
          full((1, HID1)),
          full((len(TAPS), HID1)),
          full((KREAD, HID1)),
          full((HID1, OUTC)),
          full((1, OUTC)),
      ],
      out_specs=pl.BlockSpec((SBB, OUTC), lambda sb, t: (sb, 0)),
      out_shape=jax.ShapeDtypeStruct((NB, OUTC), jnp.float32),
      scratch_shapes=[pltpu.VMEM((SBB, HID1), jnp.float32)],
  )(h0g, h1g, m2g, wl0, wr0, b0r, wl1, wr1, b1r, dwt, kdt, wp, bpr)


def kernel(x, nodes, hop1, hop2, Wl0, Wr0, b0, Wl1, Wr1, b1,
           delay_w, k_dw, Wp, bp):
  h1i = hop1.transpose(0, 2, 1).reshape(TT * N1)
  h2 = hop2.reshape(TT, NB, S1, S2).transpose(0, 2, 1, 3).reshape(TT * N1, S2)
  h2e = h2[:, 0]
  h2o = h2[:, 1]
  h0g, h1g, m2g = _sc_gather_fn()(x, nodes, h1i, h2e, h2o)
  h1g = h1g.reshape(TT, S1, NB, DF)
  m2g = m2g.reshape(TT, S1, NB, DF)
  dwt = jnp.repeat(delay_w, HID1 // GROUPS, axis=0).T
  kdt = k_dw.T
  return _tc_net(h0g, h1g, m2g, Wl0, Wr0, b0.reshape(1, -1),
                 Wl1, Wr1, b1.reshape(1, -1), dwt, kdt,
                 Wp, bp.reshape(1, -1))
